# baseline (speedup 1.0000x reference)
import functools

import jax
import jax.numpy as jnp
from jax import lax
from jax.experimental import pallas as pl
from jax.experimental.pallas import tpu as pltpu
from jax.experimental.pallas import tpu_sc as plsc

NC = 2
NS = 16
LANES = 16
NW = NC * NS

VOCAB = 1_000_000
EMB = 32
NCLS = 2

B = 16384
L = 20
TOT = B * L
PER_W = TOT // NW
BPW = B // NW
CHUNK = 128
CPW = PER_W // CHUNK

BLK = 8192
NSTEP = 31
MAXBI = VOCAB // BLK
VPAD = NSTEP * 4 * BLK


def _proj_body(taba_ref, tabb_ref, tabc_ref, tabd_ref, w_ref, b_ref,
               p0_ref, p1_ref):
    dn = (((1,), (0,)), ((), ()))
    for s, tref in enumerate((taba_ref, tabb_ref, tabc_ref, tabd_ref)):
        p = lax.dot_general(w_ref[...], tref[...], dn,
                            preferred_element_type=jnp.float32)
        p0_ref[pl.ds(s * BLK, BLK)] = p[0] + b_ref[0]
        p1_ref[pl.ds(s * BLK, BLK)] = p[1] + b_ref[1]


@functools.cache
def _build_proj():
    return pl.pallas_call(
        _proj_body,
        grid=(NSTEP,),
        in_specs=[
            pl.BlockSpec((EMB, BLK), lambda i: (0, jnp.minimum(4 * i, MAXBI))),
            pl.BlockSpec((EMB, BLK),
                         lambda i: (0, jnp.minimum(4 * i + 1, MAXBI))),
            pl.BlockSpec((EMB, BLK),
                         lambda i: (0, jnp.minimum(4 * i + 2, MAXBI))),
            pl.BlockSpec((EMB, BLK),
                         lambda i: (0, jnp.minimum(4 * i + 3, MAXBI))),
            pl.BlockSpec((NCLS, EMB), lambda i: (0, 0)),
            pl.BlockSpec((NCLS,), lambda i: (0,)),
        ],
        out_specs=[
            pl.BlockSpec((4 * BLK,), lambda i: (i,)),
            pl.BlockSpec((4 * BLK,), lambda i: (i,)),
        ],
        out_shape=[
            jax.ShapeDtypeStruct((VPAD,), jnp.float32),
            jax.ShapeDtypeStruct((VPAD,), jnp.float32),
        ],
    )


def _gather_body(xt_ref, p0_ref, p1_ref, out_ref, idxl_v, idx_v, oi_v, sem):
    wid = lax.axis_index("s") * NC + lax.axis_index("c")
    pltpu.sync_copy(xt_ref.at[:, pl.ds(wid * BPW, BPW)], idxl_v)

    iota = lax.iota(jnp.int32, LANES)

    @pl.loop(0, BPW // LANES)
    def _permute(dbg):
        src = pl.ds(dbg * LANES, LANES)
        j20 = (dbg * LANES + iota) * L
        for l in range(L):
            j = j20 + l
            plsc.store_scatter(
                idx_v, [lax.shift_right_logical(j, 7), lax.bitwise_and(j, 127)],
                idxl_v[l, src])

    @pl.loop(0, CPW)
    def _fire(c):
        pltpu.async_copy(p0_ref.at[idx_v.at[c]], oi_v.at[2 * c], sem)
        pltpu.async_copy(p1_ref.at[idx_v.at[c]], oi_v.at[2 * c + 1], sem)

    pltpu.make_async_copy(
        out_ref.at[pl.ds(0, 2 * CPW)], oi_v, sem).wait()

    pltpu.sync_copy(oi_v, out_ref.at[pl.ds(wid * 2 * CPW, 2 * CPW)])


@functools.cache
def _build_gather():
    mesh = plsc.VectorSubcoreMesh(
        core_axis_name="c", subcore_axis_name="s",
        num_cores=NC, num_subcores=NS)
    return pl.kernel(
        _gather_body,
        out_type=jax.ShapeDtypeStruct((NW * 2 * CPW, CHUNK), jnp.float32),
        mesh=mesh,
        compiler_params=pltpu.CompilerParams(
            needs_layout_passes=False, use_tc_tiling_on_sc=False),
        scratch_types=[
            pltpu.VMEM((L, BPW), jnp.int32),
            pltpu.VMEM((CPW, CHUNK), jnp.int32),
            pltpu.VMEM((2 * CPW, CHUNK), jnp.float32),
            pltpu.SemaphoreType.DMA,
        ],
    )


def kernel(x, emb_table, fc1_w, fc1_b):
    xt = x.T.astype(jnp.int32)
    tabt = emb_table.T
    p0, p1 = _build_proj()(tabt, tabt, tabt, tabt, fc1_w, fc1_b)
    out = _build_gather()(xt, p0, p1)
    return out.reshape(TOT // CHUNK, NCLS, CHUNK).transpose(0, 2, 1).reshape(TOT, NCLS)


# speedup vs baseline: 1.0030x; 1.0030x over previous
import functools

import jax
import jax.numpy as jnp
from jax import lax
from jax.experimental import pallas as pl
from jax.experimental.pallas import tpu as pltpu
from jax.experimental.pallas import tpu_sc as plsc

NC = 2
NS = 16
LANES = 16
NW = NC * NS

VOCAB = 1_000_000
EMB = 32
NCLS = 2

B = 16384
L = 20
TOT = B * L
PER_W = TOT // NW
BPW = B // NW
CHUNK = 128
CPW = PER_W // CHUNK

BLK = 16384
NSTEP = 31
VPAD = NSTEP * 2 * BLK


def _proj_body(taba_ref, tabb_ref, w_ref, b_ref, p0_ref, p1_ref):
    dn = (((1,), (0,)), ((), ()))
    for s, tref in enumerate((taba_ref, tabb_ref)):
        p = lax.dot_general(w_ref[...], tref[...], dn,
                            preferred_element_type=jnp.float32)
        p0_ref[pl.ds(s * BLK, BLK)] = p[0] + b_ref[0]
        p1_ref[pl.ds(s * BLK, BLK)] = p[1] + b_ref[1]


@functools.cache
def _build_proj():
    return pl.pallas_call(
        _proj_body,
        grid=(NSTEP,),
        in_specs=[
            pl.BlockSpec((EMB, BLK), lambda i: (0, 2 * i)),
            pl.BlockSpec((EMB, BLK), lambda i: (0, 2 * i + 1)),
            pl.BlockSpec((NCLS, EMB), lambda i: (0, 0)),
            pl.BlockSpec((NCLS,), lambda i: (0,)),
        ],
        out_specs=[
            pl.BlockSpec((2 * BLK,), lambda i: (i,)),
            pl.BlockSpec((2 * BLK,), lambda i: (i,)),
        ],
        out_shape=[
            jax.ShapeDtypeStruct((VPAD,), jnp.float32),
            jax.ShapeDtypeStruct((VPAD,), jnp.float32),
        ],
    )


def _gather_body(xt_ref, p0_ref, p1_ref, out_ref, idxl_v, idx_v, oi_v, sem):
    wid = lax.axis_index("s") * NC + lax.axis_index("c")
    pltpu.sync_copy(xt_ref.at[:, pl.ds(wid * BPW, BPW)], idxl_v)

    iota = lax.iota(jnp.int32, LANES)

    @pl.loop(0, BPW // LANES)
    def _permute(dbg):
        src = pl.ds(dbg * LANES, LANES)
        j20 = (dbg * LANES + iota) * L
        for l in range(L):
            j = j20 + l
            plsc.store_scatter(
                idx_v, [lax.shift_right_logical(j, 7), lax.bitwise_and(j, 127)],
                idxl_v[l, src])

    @pl.loop(0, CPW)
    def _fire(c):
        pltpu.async_copy(p0_ref.at[idx_v.at[c]], oi_v.at[2 * c], sem)
        pltpu.async_copy(p1_ref.at[idx_v.at[c]], oi_v.at[2 * c + 1], sem)

    pltpu.make_async_copy(
        out_ref.at[pl.ds(0, 2 * CPW)], oi_v, sem).wait()

    pltpu.sync_copy(oi_v, out_ref.at[pl.ds(wid * 2 * CPW, 2 * CPW)])


@functools.cache
def _build_gather():
    mesh = plsc.VectorSubcoreMesh(
        core_axis_name="c", subcore_axis_name="s",
        num_cores=NC, num_subcores=NS)
    return pl.kernel(
        _gather_body,
        out_type=jax.ShapeDtypeStruct((NW * 2 * CPW, CHUNK), jnp.float32),
        mesh=mesh,
        compiler_params=pltpu.CompilerParams(
            needs_layout_passes=False, use_tc_tiling_on_sc=False),
        scratch_types=[
            pltpu.VMEM((L, BPW), jnp.int32),
            pltpu.VMEM((CPW, CHUNK), jnp.int32),
            pltpu.VMEM((2 * CPW, CHUNK), jnp.float32),
            pltpu.SemaphoreType.DMA,
        ],
    )


def kernel(x, emb_table, fc1_w, fc1_b):
    xt = x.T.astype(jnp.int32)
    tabt = emb_table.T
    p0, p1 = _build_proj()(tabt, tabt, fc1_w, fc1_b)
    out = _build_gather()(xt, p0, p1)
    return out.reshape(TOT // CHUNK, NCLS, CHUNK).transpose(0, 2, 1).reshape(TOT, NCLS)
